# baseline (speedup 1.0000x reference)
import functools

import jax
import jax.numpy as jnp
from jax import lax
from jax.experimental import pallas as pl
from jax.experimental.pallas import tpu as pltpu
from jax.experimental.pallas import tpu_sc as plsc

N0, N1, N2 = 10000, 1000, 100
D = 128
E = 320000
L = 2

NC, NS = 2, 16
NW = NC * NS
EPT = E // NW
CH = 80
NCH = EPT // CH
N0A = 10240
STR0 = N0A // NS
ZR = 128

_MESH = plsc.VectorSubcoreMesh(core_axis_name="c", subcore_axis_name="s")
_SC_PARAMS = pltpu.CompilerParams(needs_layout_passes=False)
_SC_PARAMS_NT = pltpu.CompilerParams(needs_layout_passes=False,
                                     use_tc_tiling_on_sc=False)


@functools.partial(
    pl.kernel,
    out_type=jax.ShapeDtypeStruct((NW, 2, N0), jnp.float32),
    mesh=_MESH,
    compiler_params=_SC_PARAMS,
    scratch_types=[
        pltpu.VMEM((EPT,), jnp.int32),
        pltpu.VMEM((EPT,), jnp.int32),
        pltpu.VMEM((EPT,), jnp.float32),
        pltpu.VMEM((N0,), jnp.float32),
        pltpu.VMEM((N0,), jnp.float32),
    ],
)
def _deg(src_hbm, dst_hbm, w_hbm, out_hbm, srcv, dstv, wv, hout, hin):
    c = lax.axis_index("c")
    s = lax.axis_index("s")
    wid = s * NC + c
    zero16 = jnp.zeros((16,), jnp.float32)

    def _zero(i, carry):
        hout[pl.ds(i * 16, 16)] = zero16
        hin[pl.ds(i * 16, 16)] = zero16
        return carry

    lax.fori_loop(0, N0 // 16, _zero, 0)

    base = wid * EPT
    pltpu.sync_copy(src_hbm.at[pl.ds(base, EPT)], srcv)
    pltpu.sync_copy(dst_hbm.at[pl.ds(base, EPT)], dstv)
    pltpu.sync_copy(w_hbm.at[pl.ds(base, EPT)], wv)

    def _edges(i, carry):
        sl = pl.ds(i * 16, 16)
        wk = wv[sl]
        plsc.addupdate_scatter(hout, [srcv[sl]], wk)
        plsc.addupdate_scatter(hin, [dstv[sl]], wk)
        return carry

    lax.fori_loop(0, EPT // 16, _edges, 0)

    pltpu.sync_copy(hout, out_hbm.at[wid, 0])
    pltpu.sync_copy(hin, out_hbm.at[wid, 1])


@functools.partial(
    pl.kernel,
    out_type=jax.ShapeDtypeStruct((NC, N0A, D), jnp.float32),
    mesh=_MESH,
    compiler_params=_SC_PARAMS_NT,
    scratch_types=[
        pltpu.VMEM((EPT,), jnp.int32),
        pltpu.VMEM((EPT,), jnp.float32),
        pltpu.VMEM((3, CH), jnp.int32),
        pltpu.VMEM((CH, D // 2), jnp.int32),
        pltpu.VMEM((CH, D // 2), jnp.int32),
        pltpu.VMEM((CH, D // 2), jnp.int32),
        pltpu.VMEM((CH, D), jnp.float32),
        pltpu.VMEM_SHARED((N0A, D), jnp.float32),
        pltpu.SemaphoreType.DMA,
        pltpu.SemaphoreType.DMA,
        pltpu.SemaphoreType.DMA,
        pltpu.SemaphoreType.DMA,
        pltpu.SemaphoreType.DMA,
        pltpu.SemaphoreType.DMA,
    ],
)
def _agg(xws_hbm, src_hbm, dst_hbm, w_hbm, out_hbm, srcv, wv, dstv,
         buf0, buf1, buf2, fbuf, agg_sh, gs0, gs1, gs2, ds0, ds1, ds2):
    c = lax.axis_index("c")
    s = lax.axis_index("s")
    wid = s * NC + c
    base = wid * EPT
    zero16 = jnp.zeros((16,), jnp.float32)
    bufs = (buf0, buf1, buf2)
    gsems = (gs0, gs1, gs2)
    dsems = (ds0, ds1, ds2)

    def _z(i, carry):
        fbuf[i // (D // 16), pl.ds((i % (D // 16)) * 16, 16)] = zero16
        return carry

    lax.fori_loop(0, CH * D // 16, _z, 0)

    def _zs(i, carry):
        pltpu.sync_copy(fbuf, agg_sh.at[pl.ds(s * STR0 + i * CH, CH)])
        return carry

    lax.fori_loop(0, STR0 // CH, _zs, 0)

    pltpu.sync_copy(src_hbm.at[pl.ds(base, EPT)], srcv)
    pltpu.sync_copy(w_hbm.at[pl.ds(base, EPT)], wv)
    plsc.subcore_barrier()

    iota2 = lax.iota(jnp.int32, 16) * 2
    himask = jnp.full((16,), -65536, jnp.int32)

    def _gather(ci, buf, sem):
        pltpu.async_copy(xws_hbm.at[srcv.at[pl.ds(ci * CH, CH)]], buf, sem)

    def _gwait(ci, buf, sem):
        pltpu.make_async_copy(xws_hbm.at[srcv.at[pl.ds(ci * CH, CH)]], buf,
                              sem).wait()

    def _dload(ci, slot, sem):
        pltpu.async_copy(dst_hbm.at[pl.ds(base + ci * CH, CH)],
                         dstv.at[slot], sem)

    def _dwait(ci, slot, sem):
        pltpu.make_async_copy(dst_hbm.at[pl.ds(base + ci * CH, CH)],
                              dstv.at[slot], sem).wait()

    def _scale(ci, buf):
        def _body(k, carry):
            wvec = wv[pl.ds(ci * CH + k * 16, 16)]
            for lane in range(16):
                wj = jnp.full((16,), wvec[lane], jnp.float32)
                j = k * 16 + lane
                rowj = jnp.full((16,), j, jnp.int32)
                for q in range(D // 32):
                    wrd = buf[j, pl.ds(q * 16, 16)]
                    lo = plsc.bitcast(wrd << 16, jnp.float32) * wj
                    hi = plsc.bitcast(wrd & himask, jnp.float32) * wj
                    cols = iota2 + (q * 32)
                    plsc.store_scatter(fbuf, [rowj, cols], lo)
                    plsc.store_scatter(fbuf, [rowj, cols + 1], hi)
            return carry

        lax.fori_loop(0, CH // 16, _body, 0)

    def _scatter(slot):
        pltpu.sync_copy(fbuf, agg_sh.at[dstv.at[slot]], add=True)

    for r in range(3):
        _dload(r, r, dsems[r])
        _gather(r, bufs[r], gsems[r])

    def _iter(h, carry):
        ci0 = 3 * h
        for r in range(3):
            ci = ci0 + r
            _gwait(ci, bufs[r], gsems[r])
            _scale(ci, bufs[r])
            _dwait(ci, r, dsems[r])
            _scatter(r)

            @pl.when(ci + 3 < NCH)
            def _():
                _dload(ci + 3, r, dsems[r])
                _gather(ci + 3, bufs[r], gsems[r])

        return carry

    lax.fori_loop(0, NCH // 3, _iter, 0)
    for r, ci in ((0, NCH - 2), (1, NCH - 1)):
        _gwait(ci, bufs[r], gsems[r])
        _scale(ci, bufs[r])
        _dwait(ci, r, dsems[r])
        _scatter(r)

    plsc.subcore_barrier()
    pltpu.sync_copy(agg_sh.at[pl.ds(s * STR0, STR0)],
                    out_hbm.at[c, pl.ds(s * STR0, STR0)])


def _silu(x):
    return x * jax.nn.sigmoid(x)


def _prep_body(parts_ref, dd_ref):
    deg = jnp.sum(parts_ref[...], axis=0)
    safe = lax.rsqrt(jnp.maximum(deg, 1e-12))
    dd_ref[...] = jnp.where(deg > 0, safe, 0.0)


def _prep(parts):
    return pl.pallas_call(
        _prep_body,
        out_shape=jax.ShapeDtypeStruct((2, N0), jnp.float32),
    )(parts)


BN0 = 400
G0 = N0 // BN0


def _pre_body(o0_ref, w_ref, wr_ref, b_ref, dos_ref, xws_ref, xwr_ref):
    o0 = o0_ref[...]
    xw = jnp.dot(o0, w_ref[...], preferred_element_type=jnp.float32)
    xws_ref[...] = (xw * dos_ref[...]).astype(jnp.bfloat16)
    xwr_ref[...] = (
        jnp.dot(o0, wr_ref[...], preferred_element_type=jnp.float32)
        + b_ref[...]
    )


def _pre(o0, w, wr, b2d, dos):
    full = lambda i: (0, 0)
    row = lambda i: (i, 0)
    return pl.pallas_call(
        _pre_body,
        grid=(G0,),
        in_specs=[
            pl.BlockSpec((BN0, D), row),
            pl.BlockSpec((D, D), full),
            pl.BlockSpec((D, D), full),
            pl.BlockSpec((1, D), full),
            pl.BlockSpec((BN0, 1), row),
        ],
        out_specs=[
            pl.BlockSpec((BN0, D), row),
            pl.BlockSpec((BN0, D), row),
        ],
        out_shape=[
            jax.ShapeDtypeStruct((N0, D), jnp.bfloat16),
            jax.ShapeDtypeStruct((N0, D), jnp.float32),
        ],
    )(o0, w, wr, b2d, dos)


def _s1mm_body(s1_ref, o1_ref, out_ref):
    out_ref[...] = jnp.dot(s1_ref[...], o1_ref[...],
                           preferred_element_type=jnp.float32)


def _s1mm(s1, o1):
    return pl.pallas_call(
        _s1mm_body,
        grid=(G0,),
        in_specs=[
            pl.BlockSpec((BN0, N1), lambda i: (i, 0)),
            pl.BlockSpec((N1, D), lambda i: (0, 0)),
        ],
        out_specs=pl.BlockSpec((BN0, D), lambda i: (i, 0)),
        out_shape=jax.ShapeDtypeStruct((N0, D), jnp.float32),
    )(s1, o1)


def _mlp_block(a, b, w1a_ref, w1b_ref, wsa_ref, wsb_ref, w2_ref, b1_ref,
               b2_ref):
    dot = lambda x, y: jnp.dot(x, y, preferred_element_type=jnp.float32)
    h = _silu(dot(a, w1a_ref[...]) + dot(b, w1b_ref[...]) + b1_ref[...])
    return (dot(h, w2_ref[...]) + b2_ref[...] + dot(a, wsa_ref[...])
            + dot(b, wsb_ref[...]))


def _posta_body(parts_ref, xwr_ref, dis_ref, s1_ref, s1o1_ref,
                w1a_ref, w1b_ref, wsa_ref, wsb_ref, w2_ref, b1_ref, b2_ref,
                gw_ref, gwr_ref, gb_ref, dos_ref,
                xws_ref, xwrn_ref, acc_ref):
    i = pl.program_id(0)
    g = (parts_ref[0] + parts_ref[1]) * dis_ref[...] + xwr_ref[...]
    o0g = _silu(g)
    contrib = lax.dot_general(s1_ref[...], o0g, (((0,), (0,)), ((), ())),
                              preferred_element_type=jnp.float32)

    @pl.when(i == 0)
    def _():
        acc_ref[...] = jnp.zeros_like(acc_ref)

    acc_ref[...] += contrib
    out0 = _mlp_block(o0g, s1o1_ref[...], w1a_ref, w1b_ref, wsa_ref,
                      wsb_ref, w2_ref, b1_ref, b2_ref)
    xw = jnp.dot(out0, gw_ref[...], preferred_element_type=jnp.float32)
    xws_ref[...] = (xw * dos_ref[...]).astype(jnp.bfloat16)
    xwrn_ref[...] = (
        jnp.dot(out0, gwr_ref[...], preferred_element_type=jnp.float32)
        + gb_ref[...]
    )


def _posta(parts, xwr, dis, s1, s1o1, w1a, w1b, wsa, wsb, w2, b1, b2,
           gw, gwr, gb, dos):
    full = lambda i: (0, 0)
    row = lambda i: (i, 0)
    return pl.pallas_call(
        _posta_body,
        grid=(G0,),
        in_specs=[
            pl.BlockSpec((NC, BN0, D), lambda i: (0, i, 0)),
            pl.BlockSpec((BN0, D), row),
            pl.BlockSpec((BN0, 1), row),
            pl.BlockSpec((BN0, N1), row),
            pl.BlockSpec((BN0, D), row),
            pl.BlockSpec((D, D), full),
            pl.BlockSpec((D, D), full),
            pl.BlockSpec((D, D), full),
            pl.BlockSpec((D, D), full),
            pl.BlockSpec((D, D), full),
            pl.BlockSpec((1, D), full),
            pl.BlockSpec((1, D), full),
            pl.BlockSpec((D, D), full),
            pl.BlockSpec((D, D), full),
            pl.BlockSpec((1, D), full),
            pl.BlockSpec((BN0, 1), row),
        ],
        out_specs=[
            pl.BlockSpec((BN0, D), row),
            pl.BlockSpec((BN0, D), row),
            pl.BlockSpec((N1, D), full),
        ],
        out_shape=[
            jax.ShapeDtypeStruct((N0, D), jnp.bfloat16),
            jax.ShapeDtypeStruct((N0, D), jnp.float32),
            jax.ShapeDtypeStruct((N1, D), jnp.float32),
        ],
    )(parts, xwr, dis, s1, s1o1, w1a, w1b, wsa, wsb, w2, b1, b2, gw, gwr,
      gb, dos)


def _postb_body(parts_ref, xwr_ref, dis_ref, s1_ref, s1o1_ref,
                w1a_ref, w1b_ref, wsa_ref, wsb_ref, w2_ref, b1_ref, b2_ref,
                out0_ref, acc_ref):
    i = pl.program_id(0)
    g = (parts_ref[0] + parts_ref[1]) * dis_ref[...] + xwr_ref[...]
    o0g = _silu(g)
    contrib = lax.dot_general(s1_ref[...], o0g, (((0,), (0,)), ((), ())),
                              preferred_element_type=jnp.float32)

    @pl.when(i == 0)
    def _():
        acc_ref[...] = jnp.zeros_like(acc_ref)

    acc_ref[...] += contrib
    out0_ref[...] = _mlp_block(o0g, s1o1_ref[...], w1a_ref, w1b_ref,
                               wsa_ref, wsb_ref, w2_ref, b1_ref, b2_ref)


def _postb(parts, xwr, dis, s1, s1o1, w1a, w1b, wsa, wsb, w2, b1, b2):
    full = lambda i: (0, 0)
    row = lambda i: (i, 0)
    return pl.pallas_call(
        _postb_body,
        grid=(G0,),
        in_specs=[
            pl.BlockSpec((NC, BN0, D), lambda i: (0, i, 0)),
            pl.BlockSpec((BN0, D), row),
            pl.BlockSpec((BN0, 1), row),
            pl.BlockSpec((BN0, N1), row),
            pl.BlockSpec((BN0, D), row),
            pl.BlockSpec((D, D), full),
            pl.BlockSpec((D, D), full),
            pl.BlockSpec((D, D), full),
            pl.BlockSpec((D, D), full),
            pl.BlockSpec((D, D), full),
            pl.BlockSpec((1, D), full),
            pl.BlockSpec((1, D), full),
        ],
        out_specs=[
            pl.BlockSpec((BN0, D), row),
            pl.BlockSpec((N1, D), full),
        ],
        out_shape=[
            jax.ShapeDtypeStruct((N0, D), jnp.float32),
            jax.ShapeDtypeStruct((N1, D), jnp.float32),
        ],
    )(parts, xwr, dis, s1, s1o1, w1a, w1b, wsa, wsb, w2, b1, b2)


def _mid_body(o1_ref, s1to0_ref, o2_ref, sel2_ref,
              w1m1_ref, w1m2_ref, w1m3_ref, wsm1_ref, wsm2_ref, wsm3_ref,
              w21_ref, b11_ref, b21_ref,
              w1e1_ref, w1e2_ref, wse1_ref, wse2_ref, w22_ref, b12_ref,
              b22_ref, o1n_ref, o2n_ref):
    dot = lambda a, b: jnp.dot(a, b, preferred_element_type=jnp.float32)
    o1 = o1_ref[...]
    s1to0 = s1to0_ref[...]
    o2 = o2_ref[...]
    sel2 = sel2_ref[...]
    s2o2 = dot(sel2, o2)
    s2to1 = lax.dot_general(sel2, o1, (((0,), (0,)), ((), ())),
                            preferred_element_type=jnp.float32)
    h1 = _silu(dot(o1, w1m1_ref[...]) + dot(s1to0, w1m2_ref[...])
               + dot(s2o2, w1m3_ref[...]) + b11_ref[...])
    o1n_ref[...] = (dot(h1, w21_ref[...]) + b21_ref[...]
                    + dot(o1, wsm1_ref[...]) + dot(s1to0, wsm2_ref[...])
                    + dot(s2o2, wsm3_ref[...]))
    h2 = _silu(dot(o2, w1e1_ref[...]) + dot(s2to1, w1e2_ref[...])
               + b12_ref[...])
    o2n_ref[...] = (dot(h2, w22_ref[...]) + b22_ref[...]
                    + dot(o2, wse1_ref[...]) + dot(s2to1, wse2_ref[...]))


def _mid(o1, s1to0, o2, sel2, w1m, wsm, w21, b11, b21, w1e, wse, w22, b12,
         b22):
    return pl.pallas_call(
        _mid_body,
        out_shape=[
            jax.ShapeDtypeStruct((N1, D), jnp.float32),
            jax.ShapeDtypeStruct((N2, D), jnp.float32),
        ],
    )(o1, s1to0, o2, sel2,
      w1m[:D], w1m[D:2 * D], w1m[2 * D:], wsm[:D], wsm[D:2 * D], wsm[2 * D:],
      w21, b11, b21,
      w1e[:D], w1e[D:], wse[:D], wse[D:], w22, b12, b22)


def kernel(x0, x1, x2, select1, select2, edge_index, edge_weight, gcn_W,
           gcn_Wr, gcn_b, W1_edge, W1_mid, Wskip_edge, Wskip_mid, W2, b1,
           b2):
    src = edge_index[0]
    dst = edge_index[1]

    parts_deg = _deg(src, dst, edge_weight)
    dd = _prep(parts_deg)
    dos = dd[0][:, None]
    dis = dd[1][:, None]

    o1, o2 = x1, x2
    xws, xwr = _pre(x0, gcn_W[0], gcn_Wr[0], gcn_b[0][None, :], dos)
    for l in range(L):
        xws_i32 = lax.bitcast_convert_type(
            xws.reshape(N0, D // 2, 2), jnp.int32)
        parts = _agg(xws_i32, src, dst, edge_weight)
        s1o1 = _s1mm(select1, o1)
        mlp_args = (W1_edge[l, 0, :D], W1_edge[l, 0, D:],
                    Wskip_edge[l, 0, :D], Wskip_edge[l, 0, D:],
                    W2[l, 0], b1[l, 0][None, :], b2[l, 0][None, :])
        if l + 1 < L:
            xws, xwr, s1to0 = _posta(
                parts, xwr, dis, select1, s1o1, *mlp_args,
                gcn_W[l + 1], gcn_Wr[l + 1], gcn_b[l + 1][None, :], dos)
        else:
            o0, s1to0 = _postb(parts, xwr, dis, select1, s1o1, *mlp_args)
        o1, o2 = _mid(o1, s1to0, o2, select2, W1_mid[l], Wskip_mid[l],
                      W2[l, 1], b1[l, 1][None, :], b2[l, 1][None, :],
                      W1_edge[l, 1], Wskip_edge[l, 1], W2[l, 2],
                      b1[l, 2][None, :], b2[l, 2][None, :])

    return o0, o1, o2


# speedup vs baseline: 1.9290x; 1.9290x over previous
import functools

import jax
import jax.numpy as jnp
from jax import lax
from jax.experimental import pallas as pl
from jax.experimental.pallas import tpu as pltpu
from jax.experimental.pallas import tpu_sc as plsc

N0, N1, N2 = 10000, 1000, 100
D = 128
E = 320000
L = 2

NC, NS = 2, 16
NW = NC * NS
EPT = E // NW
CH = 80
NCH = EPT // CH
N0A = 10240
STR0 = N0A // NS
ZR = 128

_MESH = plsc.VectorSubcoreMesh(core_axis_name="c", subcore_axis_name="s")
_SC_PARAMS = pltpu.CompilerParams(needs_layout_passes=False)


@functools.partial(
    pl.kernel,
    out_type=jax.ShapeDtypeStruct((NW, 2, N0), jnp.float32),
    mesh=_MESH,
    compiler_params=_SC_PARAMS,
    scratch_types=[
        pltpu.VMEM((EPT,), jnp.int32),
        pltpu.VMEM((EPT,), jnp.int32),
        pltpu.VMEM((EPT,), jnp.float32),
        pltpu.VMEM((N0,), jnp.float32),
        pltpu.VMEM((N0,), jnp.float32),
    ],
)
def _deg(src_hbm, dst_hbm, w_hbm, out_hbm, srcv, dstv, wv, hout, hin):
    c = lax.axis_index("c")
    s = lax.axis_index("s")
    wid = s * NC + c
    zero16 = jnp.zeros((16,), jnp.float32)

    def _zero(i, carry):
        hout[pl.ds(i * 16, 16)] = zero16
        hin[pl.ds(i * 16, 16)] = zero16
        return carry

    lax.fori_loop(0, N0 // 16, _zero, 0)

    base = wid * EPT
    pltpu.sync_copy(src_hbm.at[pl.ds(base, EPT)], srcv)
    pltpu.sync_copy(dst_hbm.at[pl.ds(base, EPT)], dstv)
    pltpu.sync_copy(w_hbm.at[pl.ds(base, EPT)], wv)

    def _edges(i, carry):
        sl = pl.ds(i * 16, 16)
        wk = wv[sl]
        plsc.addupdate_scatter(hout, [srcv[sl]], wk)
        plsc.addupdate_scatter(hin, [dstv[sl]], wk)
        return carry

    lax.fori_loop(0, EPT // 16, _edges, 0)

    pltpu.sync_copy(hout, out_hbm.at[wid, 0])
    pltpu.sync_copy(hin, out_hbm.at[wid, 1])


@functools.partial(
    pl.kernel,
    out_type=jax.ShapeDtypeStruct((NC, N0A, D), jnp.float32),
    mesh=_MESH,
    compiler_params=_SC_PARAMS,
    scratch_types=[
        pltpu.VMEM((EPT,), jnp.int32),
        pltpu.VMEM((EPT,), jnp.float32),
        pltpu.VMEM((2, CH), jnp.int32),
        pltpu.VMEM((CH, D), jnp.float32),
        pltpu.VMEM((CH, D), jnp.float32),
        pltpu.VMEM_SHARED((N0A, D), jnp.float32),
        pltpu.SemaphoreType.DMA,
        pltpu.SemaphoreType.DMA,
        pltpu.SemaphoreType.DMA,
        pltpu.SemaphoreType.DMA,
    ],
)
def _agg(xws_hbm, src_hbm, dst_hbm, w_hbm, out_hbm, srcv, wv, dstv,
         buf_a, buf_b, agg_sh, sem_a, sem_b, sem_d0, sem_d1):
    c = lax.axis_index("c")
    s = lax.axis_index("s")
    wid = s * NC + c
    base = wid * EPT
    zero16 = jnp.zeros((16,), jnp.float32)

    def _z(i, carry):
        buf_a[i // (D // 16), pl.ds((i % (D // 16)) * 16, 16)] = zero16
        return carry

    lax.fori_loop(0, CH * D // 16, _z, 0)

    def _zs(i, carry):
        pltpu.sync_copy(buf_a, agg_sh.at[pl.ds(s * STR0 + i * CH, CH)])
        return carry

    lax.fori_loop(0, STR0 // CH, _zs, 0)

    pltpu.sync_copy(src_hbm.at[pl.ds(base, EPT)], srcv)
    pltpu.sync_copy(w_hbm.at[pl.ds(base, EPT)], wv)
    plsc.subcore_barrier()

    def _gather(ci, buf, sem):
        pltpu.async_copy(xws_hbm.at[srcv.at[pl.ds(ci * CH, CH)]], buf, sem)

    def _gwait(ci, buf, sem):
        pltpu.make_async_copy(xws_hbm.at[srcv.at[pl.ds(ci * CH, CH)]], buf,
                              sem).wait()

    def _dload(ci, slot, sem):
        pltpu.async_copy(dst_hbm.at[pl.ds(base + ci * CH, CH)],
                         dstv.at[slot], sem)

    def _dwait(ci, slot, sem):
        pltpu.make_async_copy(dst_hbm.at[pl.ds(base + ci * CH, CH)],
                              dstv.at[slot], sem).wait()

    def _scale(ci, buf):
        def _body(k, carry):
            wvec = wv[pl.ds(ci * CH + k * 16, 16)]
            for lane in range(16):
                wj = jnp.full((16,), wvec[lane], jnp.float32)
                j = k * 16 + lane
                for q in range(D // 16):
                    sl = pl.ds(q * 16, 16)
                    buf[j, sl] = buf[j, sl] * wj
            return carry

        lax.fori_loop(0, CH // 16, _body, 0)

    def _scatter(buf, slot):
        pltpu.sync_copy(buf, agg_sh.at[dstv.at[slot]], add=True)

    _dload(0, 0, sem_d0)
    _dload(1, 1, sem_d1)
    _gather(0, buf_a, sem_a)
    _gather(1, buf_b, sem_b)

    def _pair(h, carry):
        ci0 = 2 * h
        _gwait(ci0, buf_a, sem_a)
        _scale(ci0, buf_a)
        _dwait(ci0, 0, sem_d0)
        _scatter(buf_a, 0)

        @pl.when(ci0 + 2 < NCH)
        def _():
            _dload(ci0 + 2, 0, sem_d0)
            _gather(ci0 + 2, buf_a, sem_a)

        _gwait(ci0 + 1, buf_b, sem_b)
        _scale(ci0 + 1, buf_b)
        _dwait(ci0 + 1, 1, sem_d1)
        _scatter(buf_b, 1)

        @pl.when(ci0 + 3 < NCH)
        def _():
            _dload(ci0 + 3, 1, sem_d1)
            _gather(ci0 + 3, buf_b, sem_b)

        return carry

    lax.fori_loop(0, NCH // 2, _pair, 0)
    _gwait(NCH - 1, buf_a, sem_a)
    _scale(NCH - 1, buf_a)
    _dwait(NCH - 1, 0, sem_d0)
    _scatter(buf_a, 0)

    plsc.subcore_barrier()
    pltpu.sync_copy(agg_sh.at[pl.ds(s * STR0, STR0)],
                    out_hbm.at[c, pl.ds(s * STR0, STR0)])


def _silu(x):
    return x * jax.nn.sigmoid(x)


def _prep_body(parts_ref, dd_ref):
    deg = jnp.sum(parts_ref[...], axis=0)
    safe = lax.rsqrt(jnp.maximum(deg, 1e-12))
    dd_ref[...] = jnp.where(deg > 0, safe, 0.0)


def _prep(parts):
    return pl.pallas_call(
        _prep_body,
        out_shape=jax.ShapeDtypeStruct((2, N0), jnp.float32),
    )(parts)


BN0 = 400
G0 = N0 // BN0


def _pre_body(o0_ref, w_ref, wr_ref, b_ref, dos_ref, xws_ref, xwr_ref):
    o0 = o0_ref[...]
    xw = jnp.dot(o0, w_ref[...], preferred_element_type=jnp.float32)
    xws_ref[...] = xw * dos_ref[...]
    xwr_ref[...] = (
        jnp.dot(o0, wr_ref[...], preferred_element_type=jnp.float32)
        + b_ref[...]
    )


def _pre(o0, w, wr, b2d, dos):
    full = lambda i: (0, 0)
    row = lambda i: (i, 0)
    return pl.pallas_call(
        _pre_body,
        grid=(G0,),
        in_specs=[
            pl.BlockSpec((BN0, D), row),
            pl.BlockSpec((D, D), full),
            pl.BlockSpec((D, D), full),
            pl.BlockSpec((1, D), full),
            pl.BlockSpec((BN0, 1), row),
        ],
        out_specs=[
            pl.BlockSpec((BN0, D), row),
            pl.BlockSpec((BN0, D), row),
        ],
        out_shape=[
            jax.ShapeDtypeStruct((N0, D), jnp.float32),
            jax.ShapeDtypeStruct((N0, D), jnp.float32),
        ],
    )(o0, w, wr, b2d, dos)


def _s1mm_body(s1_ref, o1_ref, out_ref):
    out_ref[...] = jnp.dot(s1_ref[...], o1_ref[...],
                           preferred_element_type=jnp.float32)


def _s1mm(s1, o1):
    return pl.pallas_call(
        _s1mm_body,
        grid=(G0,),
        in_specs=[
            pl.BlockSpec((BN0, N1), lambda i: (i, 0)),
            pl.BlockSpec((N1, D), lambda i: (0, 0)),
        ],
        out_specs=pl.BlockSpec((BN0, D), lambda i: (i, 0)),
        out_shape=jax.ShapeDtypeStruct((N0, D), jnp.float32),
    )(s1, o1)


def _mlp_block(a, b, w1a_ref, w1b_ref, wsa_ref, wsb_ref, w2_ref, b1_ref,
               b2_ref):
    dot = lambda x, y: jnp.dot(x, y, preferred_element_type=jnp.float32)
    h = _silu(dot(a, w1a_ref[...]) + dot(b, w1b_ref[...]) + b1_ref[...])
    return (dot(h, w2_ref[...]) + b2_ref[...] + dot(a, wsa_ref[...])
            + dot(b, wsb_ref[...]))


def _posta_body(parts_ref, xwr_ref, dis_ref, s1_ref, s1o1_ref,
                w1a_ref, w1b_ref, wsa_ref, wsb_ref, w2_ref, b1_ref, b2_ref,
                gw_ref, gwr_ref, gb_ref, dos_ref,
                xws_ref, xwrn_ref, acc_ref):
    i = pl.program_id(0)
    g = (parts_ref[0] + parts_ref[1]) * dis_ref[...] + xwr_ref[...]
    o0g = _silu(g)
    contrib = lax.dot_general(s1_ref[...], o0g.astype(jnp.bfloat16),
                              (((0,), (0,)), ((), ())),
                              preferred_element_type=jnp.float32)

    @pl.when(i == 0)
    def _():
        acc_ref[...] = jnp.zeros_like(acc_ref)

    acc_ref[...] += contrib
    out0 = _mlp_block(o0g, s1o1_ref[...], w1a_ref, w1b_ref, wsa_ref,
                      wsb_ref, w2_ref, b1_ref, b2_ref)
    xw = jnp.dot(out0, gw_ref[...], preferred_element_type=jnp.float32)
    xws_ref[...] = xw * dos_ref[...]
    xwrn_ref[...] = (
        jnp.dot(out0, gwr_ref[...], preferred_element_type=jnp.float32)
        + gb_ref[...]
    )


def _posta(parts, xwr, dis, s1, s1o1, w1a, w1b, wsa, wsb, w2, b1, b2,
           gw, gwr, gb, dos):
    full = lambda i: (0, 0)
    row = lambda i: (i, 0)
    return pl.pallas_call(
        _posta_body,
        grid=(G0,),
        in_specs=[
            pl.BlockSpec((NC, BN0, D), lambda i: (0, i, 0)),
            pl.BlockSpec((BN0, D), row),
            pl.BlockSpec((BN0, 1), row),
            pl.BlockSpec((BN0, N1), row),
            pl.BlockSpec((BN0, D), row),
            pl.BlockSpec((D, D), full),
            pl.BlockSpec((D, D), full),
            pl.BlockSpec((D, D), full),
            pl.BlockSpec((D, D), full),
            pl.BlockSpec((D, D), full),
            pl.BlockSpec((1, D), full),
            pl.BlockSpec((1, D), full),
            pl.BlockSpec((D, D), full),
            pl.BlockSpec((D, D), full),
            pl.BlockSpec((1, D), full),
            pl.BlockSpec((BN0, 1), row),
        ],
        out_specs=[
            pl.BlockSpec((BN0, D), row),
            pl.BlockSpec((BN0, D), row),
            pl.BlockSpec((N1, D), full),
        ],
        out_shape=[
            jax.ShapeDtypeStruct((N0, D), jnp.float32),
            jax.ShapeDtypeStruct((N0, D), jnp.float32),
            jax.ShapeDtypeStruct((N1, D), jnp.float32),
        ],
    )(parts, xwr, dis, s1, s1o1, w1a, w1b, wsa, wsb, w2, b1, b2, gw, gwr,
      gb, dos)


def _postb_body(parts_ref, xwr_ref, dis_ref, s1_ref, s1o1_ref,
                w1a_ref, w1b_ref, wsa_ref, wsb_ref, w2_ref, b1_ref, b2_ref,
                out0_ref, acc_ref):
    i = pl.program_id(0)
    g = (parts_ref[0] + parts_ref[1]) * dis_ref[...] + xwr_ref[...]
    o0g = _silu(g)
    contrib = lax.dot_general(s1_ref[...], o0g.astype(jnp.bfloat16),
                              (((0,), (0,)), ((), ())),
                              preferred_element_type=jnp.float32)

    @pl.when(i == 0)
    def _():
        acc_ref[...] = jnp.zeros_like(acc_ref)

    acc_ref[...] += contrib
    out0_ref[...] = _mlp_block(o0g, s1o1_ref[...], w1a_ref, w1b_ref,
                               wsa_ref, wsb_ref, w2_ref, b1_ref, b2_ref)


def _postb(parts, xwr, dis, s1, s1o1, w1a, w1b, wsa, wsb, w2, b1, b2):
    full = lambda i: (0, 0)
    row = lambda i: (i, 0)
    return pl.pallas_call(
        _postb_body,
        grid=(G0,),
        in_specs=[
            pl.BlockSpec((NC, BN0, D), lambda i: (0, i, 0)),
            pl.BlockSpec((BN0, D), row),
            pl.BlockSpec((BN0, 1), row),
            pl.BlockSpec((BN0, N1), row),
            pl.BlockSpec((BN0, D), row),
            pl.BlockSpec((D, D), full),
            pl.BlockSpec((D, D), full),
            pl.BlockSpec((D, D), full),
            pl.BlockSpec((D, D), full),
            pl.BlockSpec((D, D), full),
            pl.BlockSpec((1, D), full),
            pl.BlockSpec((1, D), full),
        ],
        out_specs=[
            pl.BlockSpec((BN0, D), row),
            pl.BlockSpec((N1, D), full),
        ],
        out_shape=[
            jax.ShapeDtypeStruct((N0, D), jnp.float32),
            jax.ShapeDtypeStruct((N1, D), jnp.float32),
        ],
    )(parts, xwr, dis, s1, s1o1, w1a, w1b, wsa, wsb, w2, b1, b2)


def _mid_body(o1_ref, s1to0_ref, o2_ref, sel2_ref,
              w1m1_ref, w1m2_ref, w1m3_ref, wsm1_ref, wsm2_ref, wsm3_ref,
              w21_ref, b11_ref, b21_ref,
              w1e1_ref, w1e2_ref, wse1_ref, wse2_ref, w22_ref, b12_ref,
              b22_ref, o1n_ref, o2n_ref):
    dot = lambda a, b: jnp.dot(a, b, preferred_element_type=jnp.float32)
    o1 = o1_ref[...]
    s1to0 = s1to0_ref[...]
    o2 = o2_ref[...]
    sel2 = sel2_ref[...]
    s2o2 = dot(sel2, o2)
    s2to1 = lax.dot_general(sel2, o1, (((0,), (0,)), ((), ())),
                            preferred_element_type=jnp.float32)
    h1 = _silu(dot(o1, w1m1_ref[...]) + dot(s1to0, w1m2_ref[...])
               + dot(s2o2, w1m3_ref[...]) + b11_ref[...])
    o1n_ref[...] = (dot(h1, w21_ref[...]) + b21_ref[...]
                    + dot(o1, wsm1_ref[...]) + dot(s1to0, wsm2_ref[...])
                    + dot(s2o2, wsm3_ref[...]))
    h2 = _silu(dot(o2, w1e1_ref[...]) + dot(s2to1, w1e2_ref[...])
               + b12_ref[...])
    o2n_ref[...] = (dot(h2, w22_ref[...]) + b22_ref[...]
                    + dot(o2, wse1_ref[...]) + dot(s2to1, wse2_ref[...]))


def _mid(o1, s1to0, o2, sel2, w1m, wsm, w21, b11, b21, w1e, wse, w22, b12,
         b22):
    return pl.pallas_call(
        _mid_body,
        out_shape=[
            jax.ShapeDtypeStruct((N1, D), jnp.float32),
            jax.ShapeDtypeStruct((N2, D), jnp.float32),
        ],
    )(o1, s1to0, o2, sel2,
      w1m[:D], w1m[D:2 * D], w1m[2 * D:], wsm[:D], wsm[D:2 * D], wsm[2 * D:],
      w21, b11, b21,
      w1e[:D], w1e[D:], wse[:D], wse[D:], w22, b12, b22)


def kernel(x0, x1, x2, select1, select2, edge_index, edge_weight, gcn_W,
           gcn_Wr, gcn_b, W1_edge, W1_mid, Wskip_edge, Wskip_mid, W2, b1,
           b2):
    src = edge_index[0]
    dst = edge_index[1]

    parts_deg = _deg(src, dst, edge_weight)
    dd = _prep(parts_deg)
    dos = dd[0][:, None]
    dis = dd[1][:, None]

    s1b = select1.astype(jnp.bfloat16)
    o1, o2 = x1, x2
    xws, xwr = _pre(x0, gcn_W[0], gcn_Wr[0], gcn_b[0][None, :], dos)
    for l in range(L):
        parts = _agg(xws, src, dst, edge_weight)
        s1o1 = _s1mm(s1b, o1.astype(jnp.bfloat16))
        mlp_args = (W1_edge[l, 0, :D], W1_edge[l, 0, D:],
                    Wskip_edge[l, 0, :D], Wskip_edge[l, 0, D:],
                    W2[l, 0], b1[l, 0][None, :], b2[l, 0][None, :])
        if l + 1 < L:
            xws, xwr, s1to0 = _posta(
                parts, xwr, dis, s1b, s1o1, *mlp_args,
                gcn_W[l + 1], gcn_Wr[l + 1], gcn_b[l + 1][None, :], dos)
        else:
            o0, s1to0 = _postb(parts, xwr, dis, s1b, s1o1, *mlp_args)
        o1, o2 = _mid(o1, s1to0, o2, select2, W1_mid[l], Wskip_mid[l],
                      W2[l, 1], b1[l, 1][None, :], b2[l, 1][None, :],
                      W1_edge[l, 1], Wskip_edge[l, 1], W2[l, 2],
                      b1[l, 2][None, :], b2[l, 2][None, :])

    return o0, o1, o2


# speedup vs baseline: 2.1062x; 1.0919x over previous
import functools

import jax
import jax.numpy as jnp
from jax import lax
from jax.experimental import pallas as pl
from jax.experimental.pallas import tpu as pltpu
from jax.experimental.pallas import tpu_sc as plsc

N0, N1, N2 = 10000, 1000, 100
D = 128
E = 320000
L = 2

NC, NS = 2, 16
NW = NC * NS
EPT = E // NW
CH = 80
NCH = EPT // CH
N0A = 10240
STR0 = N0A // NS
ZR = 128

_MESH = plsc.VectorSubcoreMesh(core_axis_name="c", subcore_axis_name="s")
_SC_PARAMS = pltpu.CompilerParams(needs_layout_passes=False)


@functools.partial(
    pl.kernel,
    out_type=jax.ShapeDtypeStruct((NW, 2, N0), jnp.float32),
    mesh=_MESH,
    compiler_params=_SC_PARAMS,
    scratch_types=[
        pltpu.VMEM((EPT,), jnp.int32),
        pltpu.VMEM((EPT,), jnp.int32),
        pltpu.VMEM((EPT,), jnp.float32),
        pltpu.VMEM((N0,), jnp.float32),
        pltpu.VMEM((N0,), jnp.float32),
    ],
)
def _deg(src_hbm, dst_hbm, w_hbm, out_hbm, srcv, dstv, wv, hout, hin):
    c = lax.axis_index("c")
    s = lax.axis_index("s")
    wid = s * NC + c
    zero16 = jnp.zeros((16,), jnp.float32)

    def _zero(i, carry):
        hout[pl.ds(i * 16, 16)] = zero16
        hin[pl.ds(i * 16, 16)] = zero16
        return carry

    lax.fori_loop(0, N0 // 16, _zero, 0)

    base = wid * EPT
    pltpu.sync_copy(src_hbm.at[pl.ds(base, EPT)], srcv)
    pltpu.sync_copy(dst_hbm.at[pl.ds(base, EPT)], dstv)
    pltpu.sync_copy(w_hbm.at[pl.ds(base, EPT)], wv)

    def _edges(i, carry):
        sl = pl.ds(i * 16, 16)
        wk = wv[sl]
        plsc.addupdate_scatter(hout, [srcv[sl]], wk)
        plsc.addupdate_scatter(hin, [dstv[sl]], wk)
        return carry

    lax.fori_loop(0, EPT // 16, _edges, 0)

    pltpu.sync_copy(hout, out_hbm.at[wid, 0])
    pltpu.sync_copy(hin, out_hbm.at[wid, 1])


@functools.partial(
    pl.kernel,
    out_type=jax.ShapeDtypeStruct((NC, N0A, D), jnp.float32),
    mesh=_MESH,
    compiler_params=_SC_PARAMS,
    scratch_types=[
        pltpu.VMEM((EPT,), jnp.int32),
        pltpu.VMEM((3, CH), jnp.float32),
        pltpu.VMEM((3, CH), jnp.int32),
        pltpu.VMEM((CH, D), jnp.float32),
        pltpu.VMEM((CH, D), jnp.float32),
        pltpu.VMEM((CH, D), jnp.float32),
        pltpu.VMEM_SHARED((N0A, D), jnp.float32),
        pltpu.SemaphoreType.DMA,
        pltpu.SemaphoreType.DMA,
        pltpu.SemaphoreType.DMA,
        pltpu.SemaphoreType.DMA,
        pltpu.SemaphoreType.DMA,
        pltpu.SemaphoreType.DMA,
        pltpu.SemaphoreType.DMA,
        pltpu.SemaphoreType.DMA,
        pltpu.SemaphoreType.DMA,
        pltpu.SemaphoreType.DMA,
        pltpu.SemaphoreType.DMA,
        pltpu.SemaphoreType.DMA,
    ],
)
def _agg(xws_hbm, src_hbm, dst_hbm, w_hbm, out_hbm, srcv, wslots, dstv,
         buf0, buf1, buf2, agg_sh,
         gs0, gs1, gs2, ss0, ss1, ss2, ds0, ds1, ds2, ws0, ws1, ws2):
    c = lax.axis_index("c")
    s = lax.axis_index("s")
    wid = s * NC + c
    base = wid * EPT
    zero16 = jnp.zeros((16,), jnp.float32)
    bufs = (buf0, buf1, buf2)
    gsems = (gs0, gs1, gs2)
    scsems = (ss0, ss1, ss2)
    dsems = (ds0, ds1, ds2)
    wsems = (ws0, ws1, ws2)

    def _z(i, carry):
        buf0[i // (D // 16), pl.ds((i % (D // 16)) * 16, 16)] = zero16
        return carry

    lax.fori_loop(0, CH * D // 16, _z, 0)

    def _zs(i, carry):
        pltpu.sync_copy(buf0, agg_sh.at[pl.ds(s * STR0 + i * CH, CH)])
        return carry

    lax.fori_loop(0, STR0 // CH, _zs, 0)

    pltpu.sync_copy(src_hbm.at[pl.ds(base, EPT)], srcv)
    plsc.subcore_barrier()

    def _gather(ci, r):
        pltpu.async_copy(xws_hbm.at[srcv.at[pl.ds(ci * CH, CH)]], bufs[r],
                         gsems[r])

    def _gwait(ci, r):
        pltpu.make_async_copy(xws_hbm.at[srcv.at[pl.ds(ci * CH, CH)]],
                              bufs[r], gsems[r]).wait()

    def _dload(ci, r):
        pltpu.async_copy(dst_hbm.at[pl.ds(base + ci * CH, CH)],
                         dstv.at[r], dsems[r])

    def _dwait(ci, r):
        pltpu.make_async_copy(dst_hbm.at[pl.ds(base + ci * CH, CH)],
                              dstv.at[r], dsems[r]).wait()

    def _wload(ci, r):
        pltpu.async_copy(w_hbm.at[pl.ds(base + ci * CH, CH)],
                         wslots.at[r], wsems[r])

    def _wwait(ci, r):
        pltpu.make_async_copy(w_hbm.at[pl.ds(base + ci * CH, CH)],
                              wslots.at[r], wsems[r]).wait()

    def _scale(r):
        buf = bufs[r]

        def _body(k, carry):
            wvec = wslots[r, pl.ds(k * 16, 16)]
            for lane in range(16):
                wj = jnp.full((16,), wvec[lane], jnp.float32)
                j = k * 16 + lane
                for q in range(D // 16):
                    sl = pl.ds(q * 16, 16)
                    buf[j, sl] = buf[j, sl] * wj
            return carry

        lax.fori_loop(0, CH // 16, _body, 0)

    def _scatter(r):
        pltpu.async_copy(bufs[r], agg_sh.at[dstv.at[r]], scsems[r],
                         add=True)

    def _scwait(r):
        pltpu.make_async_copy(bufs[r], agg_sh.at[dstv.at[r]],
                              scsems[r]).wait()

    for r in range(2):
        _dload(r, r)
        _wload(r, r)
        _gather(r, r)

    def _iter(h, carry):
        for r in range(3):
            ci = 3 * h + r
            rn = (r + 2) % 3
            _gwait(ci, r)
            _wwait(ci, r)
            _scale(r)
            _dwait(ci, r)
            _scatter(r)

            @pl.when(ci >= 1)
            def _():
                _scwait(rn)

            _dload(ci + 2, rn)
            _wload(ci + 2, rn)
            _gather(ci + 2, rn)
        return carry

    lax.fori_loop(0, NCH // 3, _iter, 0)
    for r, ci in ((0, NCH - 2), (1, NCH - 1)):
        _gwait(ci, r)
        _wwait(ci, r)
        _scale(r)
        _dwait(ci, r)
        _scatter(r)
    for r in range(3):
        _scwait(r)

    plsc.subcore_barrier()
    pltpu.sync_copy(agg_sh.at[pl.ds(s * STR0, STR0)],
                    out_hbm.at[c, pl.ds(s * STR0, STR0)])


def _silu(x):
    return x * jax.nn.sigmoid(x)


def _prep_body(parts_ref, dd_ref):
    deg = jnp.sum(parts_ref[...], axis=0)
    safe = lax.rsqrt(jnp.maximum(deg, 1e-12))
    dd_ref[...] = jnp.where(deg > 0, safe, 0.0)


def _prep(parts):
    return pl.pallas_call(
        _prep_body,
        out_shape=jax.ShapeDtypeStruct((2, N0), jnp.float32),
    )(parts)


BN0 = 400
G0 = N0 // BN0


def _pre_body(o0_ref, w_ref, wr_ref, b_ref, dos_ref, xws_ref, xwr_ref):
    o0 = o0_ref[...]
    xw = jnp.dot(o0, w_ref[...], preferred_element_type=jnp.float32)
    xws_ref[...] = xw * dos_ref[...]
    xwr_ref[...] = (
        jnp.dot(o0, wr_ref[...], preferred_element_type=jnp.float32)
        + b_ref[...]
    )


def _pre(o0, w, wr, b2d, dos):
    full = lambda i: (0, 0)
    row = lambda i: (i, 0)
    return pl.pallas_call(
        _pre_body,
        grid=(G0,),
        in_specs=[
            pl.BlockSpec((BN0, D), row),
            pl.BlockSpec((D, D), full),
            pl.BlockSpec((D, D), full),
            pl.BlockSpec((1, D), full),
            pl.BlockSpec((BN0, 1), row),
        ],
        out_specs=[
            pl.BlockSpec((BN0, D), row),
            pl.BlockSpec((BN0, D), row),
        ],
        out_shape=[
            jax.ShapeDtypeStruct((N0, D), jnp.float32),
            jax.ShapeDtypeStruct((N0, D), jnp.float32),
        ],
    )(o0, w, wr, b2d, dos)


def _s1mm_body(s1_ref, o1_ref, out_ref):
    out_ref[...] = jnp.dot(s1_ref[...], o1_ref[...],
                           preferred_element_type=jnp.float32)


def _s1mm(s1, o1):
    return pl.pallas_call(
        _s1mm_body,
        grid=(G0,),
        in_specs=[
            pl.BlockSpec((BN0, N1), lambda i: (i, 0)),
            pl.BlockSpec((N1, D), lambda i: (0, 0)),
        ],
        out_specs=pl.BlockSpec((BN0, D), lambda i: (i, 0)),
        out_shape=jax.ShapeDtypeStruct((N0, D), jnp.float32),
    )(s1, o1)


def _mlp_block(a, b, w1a_ref, w1b_ref, wsa_ref, wsb_ref, w2_ref, b1_ref,
               b2_ref):
    dot = lambda x, y: jnp.dot(x, y, preferred_element_type=jnp.float32)
    h = _silu(dot(a, w1a_ref[...]) + dot(b, w1b_ref[...]) + b1_ref[...])
    return (dot(h, w2_ref[...]) + b2_ref[...] + dot(a, wsa_ref[...])
            + dot(b, wsb_ref[...]))


def _posta_body(parts_ref, xwr_ref, dis_ref, s1_ref, s1o1_ref,
                w1a_ref, w1b_ref, wsa_ref, wsb_ref, w2_ref, b1_ref, b2_ref,
                gw_ref, gwr_ref, gb_ref, dos_ref,
                xws_ref, xwrn_ref, acc_ref):
    i = pl.program_id(0)
    g = (parts_ref[0] + parts_ref[1]) * dis_ref[...] + xwr_ref[...]
    o0g = _silu(g)
    contrib = lax.dot_general(s1_ref[...], o0g.astype(jnp.bfloat16),
                              (((0,), (0,)), ((), ())),
                              preferred_element_type=jnp.float32)

    @pl.when(i == 0)
    def _():
        acc_ref[...] = jnp.zeros_like(acc_ref)

    acc_ref[...] += contrib
    out0 = _mlp_block(o0g, s1o1_ref[...], w1a_ref, w1b_ref, wsa_ref,
                      wsb_ref, w2_ref, b1_ref, b2_ref)
    xw = jnp.dot(out0, gw_ref[...], preferred_element_type=jnp.float32)
    xws_ref[...] = xw * dos_ref[...]
    xwrn_ref[...] = (
        jnp.dot(out0, gwr_ref[...], preferred_element_type=jnp.float32)
        + gb_ref[...]
    )


def _posta(parts, xwr, dis, s1, s1o1, w1a, w1b, wsa, wsb, w2, b1, b2,
           gw, gwr, gb, dos):
    full = lambda i: (0, 0)
    row = lambda i: (i, 0)
    return pl.pallas_call(
        _posta_body,
        grid=(G0,),
        in_specs=[
            pl.BlockSpec((NC, BN0, D), lambda i: (0, i, 0)),
            pl.BlockSpec((BN0, D), row),
            pl.BlockSpec((BN0, 1), row),
            pl.BlockSpec((BN0, N1), row),
            pl.BlockSpec((BN0, D), row),
            pl.BlockSpec((D, D), full),
            pl.BlockSpec((D, D), full),
            pl.BlockSpec((D, D), full),
            pl.BlockSpec((D, D), full),
            pl.BlockSpec((D, D), full),
            pl.BlockSpec((1, D), full),
            pl.BlockSpec((1, D), full),
            pl.BlockSpec((D, D), full),
            pl.BlockSpec((D, D), full),
            pl.BlockSpec((1, D), full),
            pl.BlockSpec((BN0, 1), row),
        ],
        out_specs=[
            pl.BlockSpec((BN0, D), row),
            pl.BlockSpec((BN0, D), row),
            pl.BlockSpec((N1, D), full),
        ],
        out_shape=[
            jax.ShapeDtypeStruct((N0, D), jnp.float32),
            jax.ShapeDtypeStruct((N0, D), jnp.float32),
            jax.ShapeDtypeStruct((N1, D), jnp.float32),
        ],
    )(parts, xwr, dis, s1, s1o1, w1a, w1b, wsa, wsb, w2, b1, b2, gw, gwr,
      gb, dos)


def _postb_body(parts_ref, xwr_ref, dis_ref, s1_ref, s1o1_ref,
                w1a_ref, w1b_ref, wsa_ref, wsb_ref, w2_ref, b1_ref, b2_ref,
                out0_ref, acc_ref):
    i = pl.program_id(0)
    g = (parts_ref[0] + parts_ref[1]) * dis_ref[...] + xwr_ref[...]
    o0g = _silu(g)
    contrib = lax.dot_general(s1_ref[...], o0g.astype(jnp.bfloat16),
                              (((0,), (0,)), ((), ())),
                              preferred_element_type=jnp.float32)

    @pl.when(i == 0)
    def _():
        acc_ref[...] = jnp.zeros_like(acc_ref)

    acc_ref[...] += contrib
    out0_ref[...] = _mlp_block(o0g, s1o1_ref[...], w1a_ref, w1b_ref,
                               wsa_ref, wsb_ref, w2_ref, b1_ref, b2_ref)


def _postb(parts, xwr, dis, s1, s1o1, w1a, w1b, wsa, wsb, w2, b1, b2):
    full = lambda i: (0, 0)
    row = lambda i: (i, 0)
    return pl.pallas_call(
        _postb_body,
        grid=(G0,),
        in_specs=[
            pl.BlockSpec((NC, BN0, D), lambda i: (0, i, 0)),
            pl.BlockSpec((BN0, D), row),
            pl.BlockSpec((BN0, 1), row),
            pl.BlockSpec((BN0, N1), row),
            pl.BlockSpec((BN0, D), row),
            pl.BlockSpec((D, D), full),
            pl.BlockSpec((D, D), full),
            pl.BlockSpec((D, D), full),
            pl.BlockSpec((D, D), full),
            pl.BlockSpec((D, D), full),
            pl.BlockSpec((1, D), full),
            pl.BlockSpec((1, D), full),
        ],
        out_specs=[
            pl.BlockSpec((BN0, D), row),
            pl.BlockSpec((N1, D), full),
        ],
        out_shape=[
            jax.ShapeDtypeStruct((N0, D), jnp.float32),
            jax.ShapeDtypeStruct((N1, D), jnp.float32),
        ],
    )(parts, xwr, dis, s1, s1o1, w1a, w1b, wsa, wsb, w2, b1, b2)


def _mid_body(o1_ref, s1to0_ref, o2_ref, sel2_ref,
              w1m1_ref, w1m2_ref, w1m3_ref, wsm1_ref, wsm2_ref, wsm3_ref,
              w21_ref, b11_ref, b21_ref,
              w1e1_ref, w1e2_ref, wse1_ref, wse2_ref, w22_ref, b12_ref,
              b22_ref, o1n_ref, o2n_ref):
    dot = lambda a, b: jnp.dot(a, b, preferred_element_type=jnp.float32)
    o1 = o1_ref[...]
    s1to0 = s1to0_ref[...]
    o2 = o2_ref[...]
    sel2 = sel2_ref[...]
    s2o2 = dot(sel2, o2)
    s2to1 = lax.dot_general(sel2, o1, (((0,), (0,)), ((), ())),
                            preferred_element_type=jnp.float32)
    h1 = _silu(dot(o1, w1m1_ref[...]) + dot(s1to0, w1m2_ref[...])
               + dot(s2o2, w1m3_ref[...]) + b11_ref[...])
    o1n_ref[...] = (dot(h1, w21_ref[...]) + b21_ref[...]
                    + dot(o1, wsm1_ref[...]) + dot(s1to0, wsm2_ref[...])
                    + dot(s2o2, wsm3_ref[...]))
    h2 = _silu(dot(o2, w1e1_ref[...]) + dot(s2to1, w1e2_ref[...])
               + b12_ref[...])
    o2n_ref[...] = (dot(h2, w22_ref[...]) + b22_ref[...]
                    + dot(o2, wse1_ref[...]) + dot(s2to1, wse2_ref[...]))


def _mid(o1, s1to0, o2, sel2, w1m, wsm, w21, b11, b21, w1e, wse, w22, b12,
         b22):
    return pl.pallas_call(
        _mid_body,
        out_shape=[
            jax.ShapeDtypeStruct((N1, D), jnp.float32),
            jax.ShapeDtypeStruct((N2, D), jnp.float32),
        ],
    )(o1, s1to0, o2, sel2,
      w1m[:D], w1m[D:2 * D], w1m[2 * D:], wsm[:D], wsm[D:2 * D], wsm[2 * D:],
      w21, b11, b21,
      w1e[:D], w1e[D:], wse[:D], wse[D:], w22, b12, b22)


def kernel(x0, x1, x2, select1, select2, edge_index, edge_weight, gcn_W,
           gcn_Wr, gcn_b, W1_edge, W1_mid, Wskip_edge, Wskip_mid, W2, b1,
           b2):
    src = edge_index[0]
    dst = edge_index[1]

    parts_deg = _deg(src, dst, edge_weight)
    dd = _prep(parts_deg)
    dos = dd[0][:, None]
    dis = dd[1][:, None]

    s1b = select1.astype(jnp.bfloat16)
    o1, o2 = x1, x2
    xws, xwr = _pre(x0, gcn_W[0], gcn_Wr[0], gcn_b[0][None, :], dos)
    for l in range(L):
        parts = _agg(xws, src, dst, edge_weight)
        s1o1 = _s1mm(s1b, o1.astype(jnp.bfloat16))
        mlp_args = (W1_edge[l, 0, :D], W1_edge[l, 0, D:],
                    Wskip_edge[l, 0, :D], Wskip_edge[l, 0, D:],
                    W2[l, 0], b1[l, 0][None, :], b2[l, 0][None, :])
        if l + 1 < L:
            xws, xwr, s1to0 = _posta(
                parts, xwr, dis, s1b, s1o1, *mlp_args,
                gcn_W[l + 1], gcn_Wr[l + 1], gcn_b[l + 1][None, :], dos)
        else:
            o0, s1to0 = _postb(parts, xwr, dis, s1b, s1o1, *mlp_args)
        o1, o2 = _mid(o1, s1to0, o2, select2, W1_mid[l], Wskip_mid[l],
                      W2[l, 1], b1[l, 1][None, :], b2[l, 1][None, :],
                      W1_edge[l, 1], Wskip_edge[l, 1], W2[l, 2],
                      b1[l, 2][None, :], b2[l, 2][None, :])

    return o0, o1, o2
